# speedup vs baseline: 3.4121x; 3.4121x over previous
import functools

import jax
import jax.numpy as jnp
from jax import lax
from jax.experimental import pallas as pl
from jax.experimental.pallas import tpu as pltpu
from jax.experimental.pallas import tpu_sc as plsc

H = 1024
FFN = 2048
E = 16
T = 4096
K = 2
P = T * K
BLK = 256
NPAD = P + E * BLK
NB = NPAD // BLK
NW = 32

G_ROWS_W = NPAD // NW
G_CG = 96
G_NCH = G_ROWS_W // G_CG

C_TOK_W = T // NW
C_CT = 32
C_NCH = C_TOK_W // C_CT


def _routing(top_experts, expert_weights):
    e_flat = top_experts.reshape(P)
    t_flat = (jnp.arange(P, dtype=jnp.int32) // K).astype(jnp.int32)
    g_flat = expert_weights.reshape(P)
    onehot = (e_flat[:, None] == jnp.arange(E, dtype=jnp.int32)[None, :]).astype(jnp.int32)
    cum = jnp.cumsum(onehot, axis=0)
    counts = cum[-1]
    rank = jnp.take_along_axis(cum, e_flat[:, None], axis=1)[:, 0] - 1
    padded = ((counts + BLK - 1) // BLK) * BLK
    pad_start = jnp.concatenate(
        [jnp.zeros(1, dtype=jnp.int32), jnp.cumsum(padded).astype(jnp.int32)])
    pos = pad_start[e_flat] + rank
    src = jnp.zeros(NPAD, jnp.int32).at[pos].set(t_flat)
    gate = jnp.zeros(NPAD, jnp.float32).at[pos].set(g_flat)
    inv = pos.reshape(T, K)
    nactive = pad_start[E] // BLK
    block_base = jnp.arange(NB, dtype=jnp.int32) * BLK
    block_expert = jnp.minimum(
        jnp.searchsorted(pad_start[1:], block_base, side="right").astype(jnp.int32),
        E - 1)
    sp = jnp.concatenate([block_expert, nactive[None].astype(jnp.int32)])
    return src, gate, inv, sp


def _sc_gather(x, src_idx):
    mesh = plsc.VectorSubcoreMesh(core_axis_name="c", subcore_axis_name="s")

    @functools.partial(
        pl.kernel, mesh=mesh,
        out_type=jax.ShapeDtypeStruct((NPAD, H), jnp.float32),
        scratch_types=[
            pltpu.VMEM((G_NCH, G_CG), jnp.int32),
            pltpu.VMEM((G_CG, H), jnp.float32),
            pltpu.SemaphoreType.DMA,
        ],
    )
    def k(x_hbm, idx_hbm, out_hbm, idx_v, buf, sem):
        wid = lax.axis_index("s") * 2 + lax.axis_index("c")
        base = wid * G_ROWS_W
        pltpu.sync_copy(idx_hbm.at[wid], idx_v)

        def body(c, carry):
            pltpu.async_copy(x_hbm.at[idx_v.at[c]], buf, sem).wait()
            pltpu.sync_copy(buf, out_hbm.at[pl.ds(base + c * G_CG, G_CG)])
            return carry

        lax.fori_loop(0, G_NCH, body, 0)

    return k(x, src_idx.reshape(NW, G_NCH, G_CG))


def _tc_grouped_mlp(xs, w1r, w2r, gate2d, sp):
    def body(sp_ref, xs_ref, w1_ref, w2_ref, g_ref, ys_ref):
        b = pl.program_id(0)

        @pl.when(b < sp_ref[NB])
        def _():
            xb = xs_ref[...].astype(jnp.bfloat16)
            w1 = w1_ref[0].astype(jnp.bfloat16)
            h = lax.dot_general(xb, w1, (((1,), (1,)), ((), ())),
                                preferred_element_type=jnp.float32)
            h = 0.5 * h * (1.0 + lax.erf(h * 0.7071067811865476))
            w2 = w2_ref[0].astype(jnp.bfloat16)
            y = lax.dot_general(h.astype(jnp.bfloat16), w2,
                                (((1,), (0,)), ((), ())),
                                preferred_element_type=jnp.float32)
            ys_ref[...] = y * g_ref[...]

    grid_spec = pltpu.PrefetchScalarGridSpec(
        num_scalar_prefetch=1,
        grid=(NB,),
        in_specs=[
            pl.BlockSpec((BLK, H), lambda b, sp: (b, 0)),
            pl.BlockSpec((1, FFN, H), lambda b, sp: (sp[b], 0, 0)),
            pl.BlockSpec((1, FFN, H), lambda b, sp: (sp[b], 0, 0)),
            pl.BlockSpec((BLK, 1), lambda b, sp: (b, 0)),
        ],
        out_specs=pl.BlockSpec((BLK, H), lambda b, sp: (b, 0)),
    )
    return pl.pallas_call(
        body,
        grid_spec=grid_spec,
        out_shape=jax.ShapeDtypeStruct((NPAD, H), jnp.float32),
    )(sp, xs, w1r, w2r, gate2d)


def _sc_combine(ys, inv0, inv1):
    mesh = plsc.VectorSubcoreMesh(core_axis_name="c", subcore_axis_name="s")

    @functools.partial(
        pl.kernel, mesh=mesh,
        out_type=jax.ShapeDtypeStruct((T, H), jnp.float32),
        scratch_types=[
            pltpu.VMEM((C_NCH, C_CT), jnp.int32),
            pltpu.VMEM((C_NCH, C_CT), jnp.int32),
            pltpu.VMEM((C_CT, H), jnp.float32),
            pltpu.VMEM((C_CT, H), jnp.float32),
            pltpu.VMEM((C_CT, H), jnp.float32),
            pltpu.SemaphoreType.DMA,
        ],
    )
    def k(ys_hbm, i0_hbm, i1_hbm, out_hbm, i0_v, i1_v, a_buf, b_buf, o_buf, sem):
        wid = lax.axis_index("s") * 2 + lax.axis_index("c")
        base = wid * C_TOK_W
        pltpu.sync_copy(i0_hbm.at[wid], i0_v)
        pltpu.sync_copy(i1_hbm.at[wid], i1_v)

        def chunk(c, carry):
            ca = pltpu.async_copy(ys_hbm.at[i0_v.at[c]], a_buf, sem)
            cb = pltpu.async_copy(ys_hbm.at[i1_v.at[c]], b_buf, sem)
            ca.wait()
            cb.wait()

            def tok(i, inner):
                for j in range(H // 16):
                    sl = pl.ds(j * 16, 16)
                    o_buf[i, sl] = a_buf[i, sl] + b_buf[i, sl]
                return inner

            lax.fori_loop(0, C_CT, tok, 0)
            pltpu.sync_copy(o_buf, out_hbm.at[pl.ds(base + c * C_CT, C_CT)])
            return carry

        lax.fori_loop(0, C_NCH, chunk, 0)

    return k(ys,
             inv0.reshape(NW, C_NCH, C_CT),
             inv1.reshape(NW, C_NCH, C_CT))


def kernel(x, scores, expert_weights, top_experts, W1, W2):
    del scores
    in_shape = x.shape
    xf = x.reshape(T, H)
    src, gate, inv, sp = _routing(top_experts, expert_weights)
    xs = _sc_gather(xf, src)
    ys = _tc_grouped_mlp(
        xs,
        W1.reshape(E, FFN, H),
        W2.reshape(E, FFN, H),
        gate.reshape(NPAD, 1),
        sp,
    )
    out = _sc_combine(ys, inv[:, 0], inv[:, 1])
    return out.reshape(in_shape)
